# Initial kernel scaffold; baseline (speedup 1.0000x reference)
import jax
import jax.numpy as jnp
from jax.experimental import pallas as pl


def kernel(feat, edge_index, W_neigh, b_neigh, W_self):
    raise NotImplementedError("write your pallas kernel here")


# speedup vs baseline: 5.5390x; 5.5390x over previous
import functools

import jax
import jax.numpy as jnp
from jax import lax
from jax.experimental import pallas as pl
from jax.experimental.pallas import tpu as pltpu
from jax.experimental.pallas import tpu_sc as plsc

N_NODES = 10000
N_EDGES = 320000
D = 128

NC = 2
NS = 16
NW = NC * NS
E_PER_W = N_EDGES // NW
CHUNK = 80
N_STEPS = E_PER_W // CHUNK
N_PAD = 10240
ROWS_PER_S = N_PAD // NS
RZ = 128


def _sc_aggregate(feat, src, dst):
    mesh = plsc.VectorSubcoreMesh(core_axis_name="c", subcore_axis_name="s")

    @functools.partial(
        pl.kernel,
        mesh=mesh,
        out_type=jax.ShapeDtypeStruct((NC * N_PAD, D), jnp.float32),
        scratch_types=[
            pltpu.VMEM((CHUNK,), jnp.int32),
            pltpu.VMEM((CHUNK,), jnp.int32),
            pltpu.VMEM((CHUNK, D), jnp.float32),
            pltpu.VMEM((RZ, D), jnp.float32),
            pltpu.VMEM_SHARED((N_PAD, D), jnp.float32),
            pltpu.SemaphoreType.DMA,
        ],
    )
    def agg_kernel(feat_hbm, src_hbm, dst_hbm, out_hbm,
                   src_v, dst_v, rows_v, buf_v, acc, sem):
        c = lax.axis_index("c")
        s = lax.axis_index("s")

        zero16 = jnp.zeros((16,), jnp.float32)

        def zero_row(i, carry):
            for j in range(D // 16):
                buf_v[i, pl.ds(j * 16, 16)] = zero16
            return carry

        lax.fori_loop(0, RZ, zero_row, 0)
        for k in range(ROWS_PER_S // RZ):
            pltpu.sync_copy(buf_v, acc.at[pl.ds(s * ROWS_PER_S + k * RZ, RZ)])
        plsc.subcore_barrier()

        ebase = (c * NS + s) * E_PER_W

        def step(t, carry):
            b = ebase + t * CHUNK
            pltpu.sync_copy(src_hbm.at[pl.ds(b, CHUNK)], src_v)
            pltpu.sync_copy(dst_hbm.at[pl.ds(b, CHUNK)], dst_v)
            pltpu.async_copy(feat_hbm.at[src_v], rows_v, sem).wait()
            pltpu.sync_copy(rows_v, acc.at[dst_v], add=True)
            return carry

        lax.fori_loop(0, N_STEPS, step, 0)
        plsc.subcore_barrier()

        for k in range(ROWS_PER_S // RZ):
            r0 = s * ROWS_PER_S + k * RZ
            pltpu.sync_copy(acc.at[pl.ds(r0, RZ)], buf_v)
            pltpu.sync_copy(buf_v, out_hbm.at[pl.ds(c * N_PAD + r0, RZ)])

    return agg_kernel(feat, src, dst)


def _tc_body(p_ref, f_ref, wn_ref, ws_ref, b_ref, o_ref):
    agg = p_ref[0] + p_ref[1]
    o_ref[...] = (
        jnp.dot(agg, wn_ref[...], preferred_element_type=jnp.float32)
        + jnp.dot(f_ref[...], ws_ref[...], preferred_element_type=jnp.float32)
        + b_ref[...]
    )


def kernel(feat, edge_index, W_neigh, b_neigh, W_self):
    src = edge_index[0].astype(jnp.int32)
    dst = edge_index[1].astype(jnp.int32)

    partials = _sc_aggregate(feat, src, dst).reshape(NC, N_PAD, D)

    B = 1000
    out = pl.pallas_call(
        _tc_body,
        grid=(N_NODES // B,),
        in_specs=[
            pl.BlockSpec((NC, B, D), lambda i: (0, i, 0)),
            pl.BlockSpec((B, D), lambda i: (i, 0)),
            pl.BlockSpec((D, D), lambda i: (0, 0)),
            pl.BlockSpec((D, D), lambda i: (0, 0)),
            pl.BlockSpec((1, D), lambda i: (0, 0)),
        ],
        out_specs=pl.BlockSpec((B, D), lambda i: (i, 0)),
        out_shape=jax.ShapeDtypeStruct((N_NODES, D), jnp.float32),
    )(partials, feat, W_neigh, W_self, b_neigh.reshape(1, D))
    return out


# speedup vs baseline: 11.1416x; 2.0115x over previous
import functools

import jax
import jax.numpy as jnp
from jax import lax
from jax.experimental import pallas as pl
from jax.experimental.pallas import tpu as pltpu
from jax.experimental.pallas import tpu_sc as plsc

N_NODES = 10000
N_EDGES = 320000
D = 128

NC = 2
NS = 16
NW = NC * NS
E_PER_W = N_EDGES // NW
CHUNK = 100
N_STEPS = E_PER_W // CHUNK
N_PAIRS = N_STEPS // 2
N_PAD = 10240
ROWS_PER_S = N_PAD // NS
RZ = 80


def _sc_aggregate(feat, edge_il):
    mesh = plsc.VectorSubcoreMesh(core_axis_name="c", subcore_axis_name="s")

    @functools.partial(
        pl.kernel,
        mesh=mesh,
        out_type=jax.ShapeDtypeStruct((NC * N_PAD, D), jnp.float32),
        scratch_types=[
            pltpu.VMEM((2, CHUNK), jnp.int32),
            pltpu.VMEM((2, CHUNK), jnp.int32),
            pltpu.VMEM((CHUNK, D), jnp.float32),
            pltpu.VMEM((CHUNK, D), jnp.float32),
            pltpu.VMEM_SHARED((N_PAD, D), jnp.float32),
            pltpu.SemaphoreType.DMA,
            pltpu.SemaphoreType.DMA,
        ],
    )
    def agg_kernel(feat_hbm, idx_hbm, out_hbm,
                   idx_a, idx_b, rows_a, rows_b, acc, gsem, isem):
        c = lax.axis_index("c")
        s = lax.axis_index("s")
        wid = c * NS + s

        pltpu.sync_copy(idx_hbm.at[wid, 0], idx_a)
        pltpu.async_copy(feat_hbm.at[idx_a.at[0]], rows_a, gsem)
        pltpu.async_copy(idx_hbm.at[wid, 1], idx_b, isem)

        zero16 = jnp.zeros((16,), jnp.float32)

        def zero_row(i, carry):
            for j in range(D // 16):
                rows_b[i, pl.ds(j * 16, 16)] = zero16
            return carry

        lax.fori_loop(0, RZ, zero_row, 0)
        zsrc = rows_b.at[pl.ds(0, RZ)]
        for k in range(ROWS_PER_S // RZ):
            pltpu.sync_copy(zsrc, acc.at[pl.ds(s * ROWS_PER_S + k * RZ, RZ)])
        plsc.subcore_barrier()

        def pair(i, carry):
            t0 = 2 * i
            pltpu.make_async_copy(
                feat_hbm.at[idx_a.at[0]], rows_a, gsem).wait()
            pltpu.make_async_copy(idx_hbm.at[wid, t0 + 1], idx_b, isem).wait()
            cp_b = pltpu.async_copy(feat_hbm.at[idx_b.at[0]], rows_b, gsem)

            pltpu.sync_copy(rows_a, acc.at[idx_a.at[1]], add=True)

            @pl.when(i + 1 < N_PAIRS)
            def _():
                pltpu.async_copy(idx_hbm.at[wid, t0 + 2], idx_a, isem)

            cp_b.wait()

            @pl.when(i + 1 < N_PAIRS)
            def _():
                pltpu.make_async_copy(idx_hbm.at[wid, t0 + 2], idx_a,
                                      isem).wait()
                pltpu.async_copy(feat_hbm.at[idx_a.at[0]], rows_a, gsem)

            pltpu.sync_copy(rows_b, acc.at[idx_b.at[1]], add=True)

            @pl.when(i + 1 < N_PAIRS)
            def _():
                pltpu.async_copy(idx_hbm.at[wid, t0 + 3], idx_b, isem)

            return carry

        lax.fori_loop(0, N_PAIRS, pair, 0)
        plsc.subcore_barrier()

        for k in range(ROWS_PER_S // RZ):
            r0 = s * ROWS_PER_S + k * RZ
            pltpu.sync_copy(acc.at[pl.ds(r0, RZ)], zsrc)
            pltpu.sync_copy(zsrc, out_hbm.at[pl.ds(c * N_PAD + r0, RZ)])

    return agg_kernel(feat, edge_il)


def _tc_body(p_ref, f_ref, wn_ref, ws_ref, b_ref, o_ref):
    agg = p_ref[0] + p_ref[1]
    o_ref[...] = (
        jnp.dot(agg, wn_ref[...], preferred_element_type=jnp.float32)
        + jnp.dot(f_ref[...], ws_ref[...], preferred_element_type=jnp.float32)
        + b_ref[...]
    )


def kernel(feat, edge_index, W_neigh, b_neigh, W_self):
    edge_il = edge_index.astype(jnp.int32) \
        .reshape(2, NW, N_STEPS, CHUNK).transpose(1, 2, 0, 3)

    partials = _sc_aggregate(feat, edge_il).reshape(NC, N_PAD, D)

    B = 1000
    out = pl.pallas_call(
        _tc_body,
        grid=(N_NODES // B,),
        in_specs=[
            pl.BlockSpec((NC, B, D), lambda i: (0, i, 0)),
            pl.BlockSpec((B, D), lambda i: (i, 0)),
            pl.BlockSpec((D, D), lambda i: (0, 0)),
            pl.BlockSpec((D, D), lambda i: (0, 0)),
            pl.BlockSpec((1, D), lambda i: (0, 0)),
        ],
        out_specs=pl.BlockSpec((B, D), lambda i: (i, 0)),
        out_shape=jax.ShapeDtypeStruct((N_NODES, D), jnp.float32),
    )(partials, feat, W_neigh, W_self, b_neigh.reshape(1, D))
    return out


# speedup vs baseline: 13.5055x; 1.2122x over previous
import functools

import jax
import jax.numpy as jnp
from jax import lax
from jax.experimental import pallas as pl
from jax.experimental.pallas import tpu as pltpu
from jax.experimental.pallas import tpu_sc as plsc

N_NODES = 10000
N_EDGES = 320000
D = 128

NC = 2
NS = 16
NW = NC * NS
E_PER_W = N_EDGES // NW
CHUNK = 125
N_STEPS = E_PER_W // CHUNK
N_QUADS = N_STEPS // 4
N_DUOS = N_QUADS // 2
N_PAD = 10240
ROWS_PER_S = N_PAD // NS
RZ = 80


def _sc_aggregate(feat, edge_il):
    mesh = plsc.VectorSubcoreMesh(core_axis_name="c", subcore_axis_name="s")

    @functools.partial(
        pl.kernel,
        mesh=mesh,
        out_type=jax.ShapeDtypeStruct((NC * N_PAD, D), jnp.float32),
        scratch_types=[
            pltpu.VMEM((4, 2, CHUNK), jnp.int32),
            pltpu.VMEM((4, 2, CHUNK), jnp.int32),
            pltpu.VMEM((CHUNK, D), jnp.float32),
            pltpu.VMEM((CHUNK, D), jnp.float32),
            pltpu.VMEM_SHARED((N_PAD, D), jnp.float32),
            pltpu.SemaphoreType.DMA,
            pltpu.SemaphoreType.DMA,
            pltpu.SemaphoreType.DMA,
        ],
    )
    def agg_kernel(feat_hbm, idx_hbm, out_hbm,
                   idx_p, idx_q, rows_a, rows_b, acc, sa, sb, isem):
        c = lax.axis_index("c")
        s = lax.axis_index("s")
        wid = c * NS + s

        pltpu.sync_copy(idx_hbm.at[wid, 0], idx_p)
        zero16 = jnp.zeros((16,), jnp.float32)

        def zero_row(i, carry):
            for j in range(D // 16):
                rows_a[i, pl.ds(j * 16, 16)] = zero16
            return carry

        lax.fori_loop(0, RZ, zero_row, 0)
        zsrc = rows_a.at[pl.ds(0, RZ)]
        for k in range(ROWS_PER_S // RZ):
            pltpu.sync_copy(zsrc, acc.at[pl.ds(s * ROWS_PER_S + k * RZ, RZ)])

        pltpu.async_copy(feat_hbm.at[idx_p.at[0, 0]], rows_a, sa)
        pltpu.async_copy(feat_hbm.at[idx_p.at[1, 0]], rows_b, sb)
        plsc.subcore_barrier()

        bufs = (rows_a, rows_b)
        sems = (sa, sb)

        def run_quad(idx_cur, idx_nxt, have_next):
            for j in range(4):
                buf, sem = bufs[j % 2], sems[j % 2]
                pltpu.make_async_copy(
                    feat_hbm.at[idx_cur.at[0, 0]], buf, sem).wait()
                pltpu.sync_copy(buf, acc.at[idx_cur.at[j, 1]], add=True)
                if j < 2:
                    pltpu.async_copy(
                        feat_hbm.at[idx_cur.at[j + 2, 0]], buf, sem)
                else:
                    @pl.when(have_next)
                    def _(j=j):
                        if j == 2:
                            pltpu.make_async_copy(
                                idx_hbm.at[wid, 0], idx_nxt, isem).wait()
                        pltpu.async_copy(
                            feat_hbm.at[idx_nxt.at[j - 2, 0]], buf, sem)

        def duo(k, carry):
            q0 = 2 * k
            pltpu.async_copy(idx_hbm.at[wid, q0 + 1], idx_q, isem)
            run_quad(idx_p, idx_q, q0 + 1 < N_QUADS)

            @pl.when(q0 + 2 < N_QUADS)
            def _():
                pltpu.async_copy(idx_hbm.at[wid, q0 + 2], idx_p, isem)

            run_quad(idx_q, idx_p, q0 + 2 < N_QUADS)
            return carry

        lax.fori_loop(0, N_DUOS, duo, 0)
        plsc.subcore_barrier()

        for k in range(ROWS_PER_S // RZ):
            r0 = s * ROWS_PER_S + k * RZ
            pltpu.sync_copy(acc.at[pl.ds(r0, RZ)], zsrc)
            pltpu.sync_copy(zsrc, out_hbm.at[pl.ds(c * N_PAD + r0, RZ)])

    return agg_kernel(feat, edge_il)


def _tc_body(p_ref, f_ref, wn_ref, ws_ref, b_ref, o_ref):
    agg = p_ref[0] + p_ref[1]
    o_ref[...] = (
        jnp.dot(agg, wn_ref[...], preferred_element_type=jnp.float32)
        + jnp.dot(f_ref[...], ws_ref[...], preferred_element_type=jnp.float32)
        + b_ref[...]
    )


def kernel(feat, edge_index, W_neigh, b_neigh, W_self):
    edge_il = edge_index.astype(jnp.int32) \
        .reshape(2, NW, N_QUADS, 4, CHUNK).transpose(1, 2, 3, 0, 4)

    partials = _sc_aggregate(feat, edge_il).reshape(NC, N_PAD, D)

    B = 1000
    out = pl.pallas_call(
        _tc_body,
        grid=(N_NODES // B,),
        in_specs=[
            pl.BlockSpec((NC, B, D), lambda i: (0, i, 0)),
            pl.BlockSpec((B, D), lambda i: (i, 0)),
            pl.BlockSpec((D, D), lambda i: (0, 0)),
            pl.BlockSpec((D, D), lambda i: (0, 0)),
            pl.BlockSpec((1, D), lambda i: (0, 0)),
        ],
        out_specs=pl.BlockSpec((B, D), lambda i: (i, 0)),
        out_shape=jax.ShapeDtypeStruct((N_NODES, D), jnp.float32),
    )(partials, feat, W_neigh, W_self, b_neigh.reshape(1, D))
    return out
